# Initial kernel scaffold; baseline (speedup 1.0000x reference)
import jax
import jax.numpy as jnp
from jax.experimental import pallas as pl


def kernel(x, edge_index, forward_level, backward_level, forward_index, W_emd, b_emd, W_aggr, b_aggr, W_ih, W_hh, b_ih, b_hh, W1, b1, W2, b2):
    raise NotImplementedError("write your pallas kernel here")


# speedup vs baseline: 4.1237x; 4.1237x over previous
import functools

import jax
import jax.numpy as jnp
from jax import lax
from jax.experimental import pallas as pl
from jax.experimental.pallas import tpu as pltpu
from jax.experimental.pallas import tpu_sc as plsc

N = 10000
E = 320000
F = 128
H = 128
L = 8
G3 = 3 * H
MLP_DIM = 128

NC = 2
NS = 16
LANES = 16

C = 128
RC = 32
NBLK = 512
NP = 10240
NB = NP // NBLK
NMID = 5120
HB = NMID // NBLK
NP2 = NMID + 64
TRASH = NP2 - 8
E_PAD = E + 1024

_f32 = jnp.float32
_i32 = jnp.int32


def _sc_level_body(params, srcs, dsts0, dsts1, hmat, zrow, zdeg, ones_h,
                   s_out, deg_out,
                   params_v, sidx_v, didx_v, rows_v, ones_v, zrow_v, zdeg_v,
                   cp_v, cpd_v, acc, dacc, sem):
    cid = lax.axis_index("c")
    sid = lax.axis_index("s")

    pltpu.sync_copy(params, params_v)
    pltpu.sync_copy(zrow, zrow_v)
    pltpu.sync_copy(zdeg, zdeg_v)
    pltpu.sync_copy(ones_h, ones_v)

    pv = params_v[...]

    def getp(i0, i1):
        return jnp.where(cid == 0, pv[i0], pv[i1])

    e_lo = getp(0, 6)
    n_ech = getp(1, 7)
    ek = getp(2, 8)
    n_lo = getp(3, 9)
    n_rch = getp(4, 10)
    n_csub = getp(5, 11)

    def zbody(j, carry):
        k = j * NS + sid

        @pl.when(k < n_rch)
        def _():
            r = pl.multiple_of(n_lo + k * RC, 8)
            pltpu.sync_copy(zrow_v, acc.at[pl.ds(r, RC)])
            pltpu.sync_copy(zdeg_v, dacc.at[pl.ds(r, RC)])
        return carry

    lax.fori_loop(0, n_csub, zbody, 0)
    plsc.subcore_barrier()

    def ebody(j, carry):
        k = j * NS + sid

        @pl.when(k < ek)
        def _():
            base = pl.multiple_of(e_lo + k * C, 8)
            pltpu.sync_copy(srcs.at[pl.ds(base, C)], sidx_v)

            @pl.when(cid == 0)
            def _():
                pltpu.sync_copy(dsts0.at[pl.ds(base, C)], didx_v)

            @pl.when(cid == 1)
            def _():
                pltpu.sync_copy(dsts1.at[pl.ds(base, C)], didx_v)

            pltpu.async_copy(hmat.at[sidx_v], rows_v, sem).wait()
            pltpu.sync_copy(rows_v, acc.at[didx_v], add=True)
            pltpu.sync_copy(ones_v, dacc.at[didx_v], add=True)
        return carry

    lax.fori_loop(0, n_ech, ebody, 0)
    plsc.subcore_barrier()

    def cbody(j, carry):
        k = j * NS + sid

        @pl.when(k < n_rch)
        def _():
            r = pl.multiple_of(n_lo + k * RC, 8)
            ro = pl.multiple_of(cid * NP2 + r, 8)
            pltpu.sync_copy(acc.at[pl.ds(r, RC)], cp_v)
            pltpu.sync_copy(cp_v, s_out.at[pl.ds(ro, RC)])
            pltpu.sync_copy(dacc.at[pl.ds(r, RC)], cpd_v)
            pltpu.sync_copy(cpd_v, deg_out.at[pl.ds(ro, RC)])
        return carry

    lax.fori_loop(0, n_csub, cbody, 0)


_sc_level = functools.partial(
    pl.kernel,
    out_type=(
        jax.ShapeDtypeStruct((NC * NP2, H), _f32),
        jax.ShapeDtypeStruct((NC * NP2, LANES), _f32),
    ),
    mesh=plsc.VectorSubcoreMesh(core_axis_name="c", subcore_axis_name="s"),
    compiler_params=pltpu.CompilerParams(use_tc_tiling_on_sc=False),
    scratch_types=(
        pltpu.VMEM((LANES,), _i32),
        pltpu.VMEM((C,), _i32),
        pltpu.VMEM((C,), _i32),
        pltpu.VMEM((C, H), _f32),
        pltpu.VMEM((C, LANES), _f32),
        pltpu.VMEM((RC, H), _f32),
        pltpu.VMEM((RC, LANES), _f32),
        pltpu.VMEM((RC, H), _f32),
        pltpu.VMEM((RC, LANES), _f32),
        pltpu.VMEM_SHARED((NP2, H), _f32),
        pltpu.VMEM_SHARED((NP2, LANES), _f32),
        pltpu.SemaphoreType.DMA,
    ),
)(_sc_level_body)


def _gblk(i, s):
    return s[0] + jnp.maximum(jnp.minimum(i, s[1] - 1), 0)


def _rowmap(i, s):
    return (_gblk(i, s), 0)


def _lomap(i, s):
    return (jnp.minimum(_gblk(i, s), HB - 1), 0)


def _himap(i, s):
    return (jnp.maximum(_gblk(i, s) - HB, 0), 0)


def _zeromap(i, s):
    return (0, 0)


def _tc_update_body(s_ref, slo, shi, dlo, dhi, xp, hin, gh0, h0r, wat, wimt,
                    bag, out):
    i = pl.program_id(0)
    nblk = s_ref[1]
    n_lo = s_ref[2]
    n_hi = s_ref[3]

    @pl.when(jnp.logical_and(nblk == 0, i == 0))
    def _():
        out[...] = hin[...]

    @pl.when(i < nblk)
    def _():
        gblk = s_ref[0] + jnp.minimum(i, nblk - 1)
        use_hi = gblk >= HB
        svals = jnp.where(use_hi, shi[...], slo[...])
        deg = jnp.where(use_hi, dhi[:, 0:1], dlo[:, 0:1])
        msg = jnp.dot(svals, wat[...], preferred_element_type=_f32,
                      precision=lax.Precision.HIGHEST) + deg * bag[...]
        gi = jnp.dot(msg, wimt[...], preferred_element_type=_f32,
                     precision=lax.Precision.HIGHEST) + xp[...]
        g0 = gh0[...]
        r = jax.nn.sigmoid(gi[:, :H] + g0[:, :H])
        z = jax.nn.sigmoid(gi[:, H:2 * H] + g0[:, H:2 * H])
        n_ = jnp.tanh(gi[:, 2 * H:] + r * g0[:, 2 * H:])
        hnew = (1.0 - z) * n_ + z * h0r[...]
        rows = gblk * NBLK + lax.broadcasted_iota(_i32, (NBLK, 1), 0)
        mask = jnp.logical_and(rows >= n_lo, rows < n_hi)
        out[...] = jnp.where(mask, hnew, hin[...])


_tc_update = pl.pallas_call(
    _tc_update_body,
    grid_spec=pltpu.PrefetchScalarGridSpec(
        num_scalar_prefetch=1,
        grid=(NB,),
        in_specs=[
            pl.BlockSpec((NBLK, H), _lomap),
            pl.BlockSpec((NBLK, H), _himap),
            pl.BlockSpec((NBLK, LANES), _lomap),
            pl.BlockSpec((NBLK, LANES), _himap),
            pl.BlockSpec((NBLK, G3), _rowmap),
            pl.BlockSpec((NBLK, H), _rowmap),
            pl.BlockSpec((1, G3), _zeromap),
            pl.BlockSpec((1, H), _zeromap),
            pl.BlockSpec((H, H), _zeromap),
            pl.BlockSpec((H, G3), _zeromap),
            pl.BlockSpec((1, H), _zeromap),
        ],
        out_specs=pl.BlockSpec((NBLK, H), _rowmap),
    ),
    out_shape=jax.ShapeDtypeStruct((NP, H), _f32),
    input_output_aliases={6: 0},
)


def _tc_pre_body(xc, wxt, bih, h0r, wht, bhh, xp_out, gh0_out):
    xp_out[...] = jnp.dot(xc[...], wxt[...], preferred_element_type=_f32,
                          precision=lax.Precision.HIGHEST) + bih[...]

    @pl.when(pl.program_id(0) == 0)
    def _():
        gh0_out[...] = jnp.dot(h0r[...], wht[...], preferred_element_type=_f32,
                               precision=lax.Precision.HIGHEST) + bhh[...]


_tc_pre = pl.pallas_call(
    _tc_pre_body,
    grid=(NB,),
    in_specs=[
        pl.BlockSpec((NBLK, F), lambda i: (i, 0)),
        pl.BlockSpec((F, G3), lambda i: (0, 0)),
        pl.BlockSpec((1, G3), lambda i: (0, 0)),
        pl.BlockSpec((1, H), lambda i: (0, 0)),
        pl.BlockSpec((H, G3), lambda i: (0, 0)),
        pl.BlockSpec((1, G3), lambda i: (0, 0)),
    ],
    out_specs=[
        pl.BlockSpec((NBLK, G3), lambda i: (i, 0)),
        pl.BlockSpec((1, G3), lambda i: (0, 0)),
    ],
    out_shape=[
        jax.ShapeDtypeStruct((NP, G3), _f32),
        jax.ShapeDtypeStruct((1, G3), _f32),
    ],
)


def _tc_mlp_body(h_ref, w1, b1r, w2, b2r, o_ref):
    a = jnp.dot(h_ref[...], w1[...], preferred_element_type=_f32,
                precision=lax.Precision.HIGHEST) + b1r[...]
    a = jnp.maximum(a, 0.0)
    o_ref[...] = jnp.dot(a, w2[...], preferred_element_type=_f32,
                         precision=lax.Precision.HIGHEST) + b2r[...]


_tc_mlp = pl.pallas_call(
    _tc_mlp_body,
    grid=(NB,),
    in_specs=[
        pl.BlockSpec((NBLK, H), lambda i: (i, 0)),
        pl.BlockSpec((H, MLP_DIM), lambda i: (0, 0)),
        pl.BlockSpec((1, MLP_DIM), lambda i: (0, 0)),
        pl.BlockSpec((MLP_DIM, 128), lambda i: (0, 0)),
        pl.BlockSpec((1, 128), lambda i: (0, 0)),
    ],
    out_specs=pl.BlockSpec((NBLK, 128), lambda i: (i, 0)),
    out_shape=jax.ShapeDtypeStruct((NP, 128), _f32),
)


def _cdiv(a, b):
    return (a + b - 1) // b


def _align8(a):
    return (a // 8) * 8


def kernel(x, edge_index, forward_level, backward_level, forward_index,
           W_emd, b_emd, W_aggr, b_aggr, W_ih, W_hh, b_ih, b_hh,
           W1, b1, W2, b2):
    src = edge_index[0]
    dst = edge_index[1]
    node_order = jnp.argsort(forward_level)
    noff = jnp.searchsorted(forward_level[node_order],
                            jnp.arange(L + 1, dtype=_i32)).astype(_i32)
    rank = jnp.zeros((N,), _i32).at[node_order].set(jnp.arange(N, dtype=_i32))
    dsr = rank[dst]
    order = jnp.argsort(dsr)
    dsr_s = dsr[order]
    eoff = jnp.searchsorted(dsr_s, noff).astype(_i32)
    esplit = jnp.searchsorted(dsr_s, NMID).astype(_i32)
    src_p = jnp.concatenate([rank[src][order],
                             jnp.zeros((E_PAD - E,), _i32)])
    dl0 = jnp.where(dsr_s < NMID, dsr_s, TRASH)
    dl1 = jnp.where(dsr_s >= NMID, dsr_s - NMID, TRASH)
    pad_tr = jnp.full((E_PAD - E,), TRASH, _i32)
    dst_p0 = jnp.concatenate([dl0, pad_tr])
    dst_p1 = jnp.concatenate([dl1, pad_tr])

    h0 = (W_emd[0] + b_emd).astype(_f32)
    h0r = h0[None, :]
    wat = W_aggr.T
    wimt = W_ih[:, :H].T
    wxt = W_ih[:, H:].T
    wht = W_hh.T
    bih = b_ih[None, :]
    bhh = b_hh[None, :]
    bag = b_aggr[None, :]
    b1r = b1[None, :]
    w2p = jnp.pad(W2, ((0, 0), (0, 128 - W2.shape[1])))
    b2r = jnp.pad(b2[None, :], ((0, 0), (0, 128 - b2.shape[0])))

    x_c = jnp.concatenate([x[node_order], jnp.zeros((NP - N, F), _f32)])

    zrow = jnp.zeros((RC, H), _f32)
    zdeg = jnp.zeros((RC, LANES), _f32)
    ones_h = jnp.ones((C, LANES), _f32)

    sc_params = []
    tc_params = []
    for l in range(1, L):
        n_lo_raw, n_hi = noff[l], noff[l + 1]
        e_a, e_c = eoff[l], eoff[l + 1]
        e_b = jnp.clip(esplit, e_a, e_c)
        lanes = []
        for (elo_raw, ehi, glo, ghi) in (
                (e_a, e_b, jnp.minimum(n_lo_raw, NMID), jnp.minimum(n_hi, NMID)),
                (e_b, e_c, jnp.maximum(n_lo_raw, NMID) - NMID,
                 jnp.maximum(n_hi, NMID) - NMID)):
            elo = _align8(elo_raw)
            ek = jnp.maximum(_cdiv(ehi - elo, C), 0)
            zlo = _align8(glo)
            nrch = jnp.maximum(_cdiv(ghi - zlo, RC), 0)
            lanes += [elo, _cdiv(ek, NS), ek, zlo, nrch, _cdiv(nrch, NS)]
        p = jnp.stack(lanes + [jnp.int32(0)] * (LANES - len(lanes))).astype(_i32)
        sc_params.append(p)
        first_blk = n_lo_raw // NBLK
        nblk = jnp.where(n_hi > n_lo_raw, (n_hi - 1) // NBLK - first_blk + 1, 0)
        tc_params.append(jnp.stack([first_blk, nblk, n_lo_raw, n_hi]).astype(_i32))

    xp, gh0 = _tc_pre(x_c, wxt, bih, h0r, wht, bhh)
    h_c = jnp.broadcast_to(h0r, (NP, H)) * jnp.ones((NP, 1), _f32)
    for i in range(L - 1):
        s_out, deg_out = _sc_level(sc_params[i], src_p, dst_p0, dst_p1, h_c,
                                   zrow, zdeg, ones_h)
        h_c = _tc_update(tc_params[i], s_out[:NP2], s_out[NP2:],
                         deg_out[:NP2], deg_out[NP2:], xp, h_c,
                         gh0, h0r, wat, wimt, bag)
    preds_full = _tc_mlp(h_c, W1, b1r, w2p, b2r)
    return preds_full[rank, :1]


# speedup vs baseline: 4.1462x; 1.0055x over previous
import functools

import jax
import jax.numpy as jnp
from jax import lax
from jax.experimental import pallas as pl
from jax.experimental.pallas import tpu as pltpu
from jax.experimental.pallas import tpu_sc as plsc

N = 10000
E = 320000
F = 128
H = 128
L = 8
G3 = 3 * H
MLP_DIM = 128

NC = 2
NS = 16
LANES = 16

C = 128
RC = 32
NBLK = 512
NP = 10240
NB = NP // NBLK
NMID = 5120
HB = NMID // NBLK
NP2 = NMID + 64
TRASH = NP2 - 8
E_PAD = E + 1024

_f32 = jnp.float32
_i32 = jnp.int32


def _sc_level_body(params, srcs, dsts0, dsts1, hmat, zrow, zdeg, ones_h,
                   s_out, deg_out,
                   params_v, sidx_v, didx_v, rows_v, ones_v, zrow_v, zdeg_v,
                   cp_v, cpd_v, acc, dacc, sem):
    cid = lax.axis_index("c")
    sid = lax.axis_index("s")

    pltpu.sync_copy(params, params_v)
    pltpu.sync_copy(zrow, zrow_v)
    pltpu.sync_copy(zdeg, zdeg_v)
    pltpu.sync_copy(ones_h, ones_v)

    pv = params_v[...]

    def getp(i0, i1):
        return jnp.where(cid == 0, pv[i0], pv[i1])

    e_lo = getp(0, 6)
    n_ech = getp(1, 7)
    ek = getp(2, 8)
    n_lo = getp(3, 9)
    n_rch = getp(4, 10)
    n_csub = getp(5, 11)

    def zbody(j, carry):
        k = j * NS + sid

        @pl.when(k < n_rch)
        def _():
            r = pl.multiple_of(n_lo + k * RC, 8)
            pltpu.sync_copy(zrow_v, acc.at[pl.ds(r, RC)])
            pltpu.sync_copy(zdeg_v, dacc.at[pl.ds(r, RC)])
        return carry

    lax.fori_loop(0, n_csub, zbody, 0)
    plsc.subcore_barrier()

    def ebody(j, carry):
        k = j * NS + sid

        @pl.when(k < ek)
        def _():
            base = pl.multiple_of(e_lo + k * C, 8)
            pltpu.sync_copy(srcs.at[pl.ds(base, C)], sidx_v)

            @pl.when(cid == 0)
            def _():
                pltpu.sync_copy(dsts0.at[pl.ds(base, C)], didx_v)

            @pl.when(cid == 1)
            def _():
                pltpu.sync_copy(dsts1.at[pl.ds(base, C)], didx_v)

            pltpu.async_copy(hmat.at[sidx_v], rows_v, sem).wait()
            pltpu.sync_copy(rows_v, acc.at[didx_v], add=True)
            pltpu.sync_copy(ones_v, dacc.at[didx_v], add=True)
        return carry

    lax.fori_loop(0, n_ech, ebody, 0)
    plsc.subcore_barrier()

    def cbody(j, carry):
        k = j * NS + sid

        @pl.when(k < n_rch)
        def _():
            r = pl.multiple_of(n_lo + k * RC, 8)
            ro = pl.multiple_of(cid * NP2 + r, 8)
            pltpu.sync_copy(acc.at[pl.ds(r, RC)], cp_v)
            pltpu.sync_copy(cp_v, s_out.at[pl.ds(ro, RC)])
            pltpu.sync_copy(dacc.at[pl.ds(r, RC)], cpd_v)
            pltpu.sync_copy(cpd_v, deg_out.at[pl.ds(ro, RC)])
        return carry

    lax.fori_loop(0, n_csub, cbody, 0)


_sc_level = functools.partial(
    pl.kernel,
    out_type=(
        jax.ShapeDtypeStruct((NC * NP2, H), _f32),
        jax.ShapeDtypeStruct((NC * NP2, LANES), _f32),
    ),
    mesh=plsc.VectorSubcoreMesh(core_axis_name="c", subcore_axis_name="s"),
    compiler_params=pltpu.CompilerParams(use_tc_tiling_on_sc=False),
    scratch_types=(
        pltpu.VMEM((LANES,), _i32),
        pltpu.VMEM((C,), _i32),
        pltpu.VMEM((C,), _i32),
        pltpu.VMEM((C, H), _f32),
        pltpu.VMEM((C, LANES), _f32),
        pltpu.VMEM((RC, H), _f32),
        pltpu.VMEM((RC, LANES), _f32),
        pltpu.VMEM((RC, H), _f32),
        pltpu.VMEM((RC, LANES), _f32),
        pltpu.VMEM_SHARED((NP2, H), _f32),
        pltpu.VMEM_SHARED((NP2, LANES), _f32),
        pltpu.SemaphoreType.DMA,
    ),
)(_sc_level_body)


def _gblk(i, s):
    return s[0] + jnp.maximum(jnp.minimum(i, s[1] - 1), 0)


def _rowmap(i, s):
    return (_gblk(i, s), 0)


def _lomap(i, s):
    return (jnp.minimum(_gblk(i, s), HB - 1), 0)


def _himap(i, s):
    return (jnp.maximum(_gblk(i, s) - HB, 0), 0)


def _zeromap(i, s):
    return (0, 0)


def _tc_update_body(s_ref, slo, shi, dlo, dhi, xp, hin, gh0, h0r, wat, wimt,
                    bag, out):
    i = pl.program_id(0)
    nblk = s_ref[1]
    n_lo = s_ref[2]
    n_hi = s_ref[3]

    @pl.when(jnp.logical_and(nblk == 0, i == 0))
    def _():
        out[...] = hin[...]

    @pl.when(i < nblk)
    def _():
        gblk = s_ref[0] + jnp.minimum(i, nblk - 1)
        use_hi = gblk >= HB
        svals = jnp.where(use_hi, shi[...], slo[...])
        deg = jnp.where(use_hi, dhi[:, 0:1], dlo[:, 0:1])
        msg = jnp.dot(svals, wat[...], preferred_element_type=_f32) + deg * bag[...]
        gi = jnp.dot(msg, wimt[...], preferred_element_type=_f32) + xp[...]
        g0 = gh0[...]
        r = jax.nn.sigmoid(gi[:, :H] + g0[:, :H])
        z = jax.nn.sigmoid(gi[:, H:2 * H] + g0[:, H:2 * H])
        n_ = jnp.tanh(gi[:, 2 * H:] + r * g0[:, 2 * H:])
        hnew = (1.0 - z) * n_ + z * h0r[...]
        rows = gblk * NBLK + lax.broadcasted_iota(_i32, (NBLK, 1), 0)
        mask = jnp.logical_and(rows >= n_lo, rows < n_hi)
        out[...] = jnp.where(mask, hnew, hin[...])


_tc_update = pl.pallas_call(
    _tc_update_body,
    grid_spec=pltpu.PrefetchScalarGridSpec(
        num_scalar_prefetch=1,
        grid=(NB,),
        in_specs=[
            pl.BlockSpec((NBLK, H), _lomap),
            pl.BlockSpec((NBLK, H), _himap),
            pl.BlockSpec((NBLK, LANES), _lomap),
            pl.BlockSpec((NBLK, LANES), _himap),
            pl.BlockSpec((NBLK, G3), _rowmap),
            pl.BlockSpec((NBLK, H), _rowmap),
            pl.BlockSpec((1, G3), _zeromap),
            pl.BlockSpec((1, H), _zeromap),
            pl.BlockSpec((H, H), _zeromap),
            pl.BlockSpec((H, G3), _zeromap),
            pl.BlockSpec((1, H), _zeromap),
        ],
        out_specs=pl.BlockSpec((NBLK, H), _rowmap),
    ),
    out_shape=jax.ShapeDtypeStruct((NP, H), _f32),
    input_output_aliases={6: 0},
)


def _tc_pre_body(xc, wxt, bih, h0r, wht, bhh, xp_out, gh0_out):
    xp_out[...] = jnp.dot(xc[...], wxt[...], preferred_element_type=_f32) + bih[...]

    @pl.when(pl.program_id(0) == 0)
    def _():
        gh0_out[...] = jnp.dot(h0r[...], wht[...], preferred_element_type=_f32) + bhh[...]


_tc_pre = pl.pallas_call(
    _tc_pre_body,
    grid=(NB,),
    in_specs=[
        pl.BlockSpec((NBLK, F), lambda i: (i, 0)),
        pl.BlockSpec((F, G3), lambda i: (0, 0)),
        pl.BlockSpec((1, G3), lambda i: (0, 0)),
        pl.BlockSpec((1, H), lambda i: (0, 0)),
        pl.BlockSpec((H, G3), lambda i: (0, 0)),
        pl.BlockSpec((1, G3), lambda i: (0, 0)),
    ],
    out_specs=[
        pl.BlockSpec((NBLK, G3), lambda i: (i, 0)),
        pl.BlockSpec((1, G3), lambda i: (0, 0)),
    ],
    out_shape=[
        jax.ShapeDtypeStruct((NP, G3), _f32),
        jax.ShapeDtypeStruct((1, G3), _f32),
    ],
)


def _tc_mlp_body(h_ref, w1, b1r, w2, b2r, o_ref):
    a = jnp.dot(h_ref[...], w1[...], preferred_element_type=_f32) + b1r[...]
    a = jnp.maximum(a, 0.0)
    o_ref[...] = jnp.dot(a, w2[...], preferred_element_type=_f32) + b2r[...]


_tc_mlp = pl.pallas_call(
    _tc_mlp_body,
    grid=(NB,),
    in_specs=[
        pl.BlockSpec((NBLK, H), lambda i: (i, 0)),
        pl.BlockSpec((H, MLP_DIM), lambda i: (0, 0)),
        pl.BlockSpec((1, MLP_DIM), lambda i: (0, 0)),
        pl.BlockSpec((MLP_DIM, 128), lambda i: (0, 0)),
        pl.BlockSpec((1, 128), lambda i: (0, 0)),
    ],
    out_specs=pl.BlockSpec((NBLK, 128), lambda i: (i, 0)),
    out_shape=jax.ShapeDtypeStruct((NP, 128), _f32),
)


def _cdiv(a, b):
    return (a + b - 1) // b


def _align8(a):
    return (a // 8) * 8


def kernel(x, edge_index, forward_level, backward_level, forward_index,
           W_emd, b_emd, W_aggr, b_aggr, W_ih, W_hh, b_ih, b_hh,
           W1, b1, W2, b2):
    src = edge_index[0]
    dst = edge_index[1]
    node_order = jnp.argsort(forward_level)
    noff = jnp.searchsorted(forward_level[node_order],
                            jnp.arange(L + 1, dtype=_i32)).astype(_i32)
    rank = jnp.zeros((N,), _i32).at[node_order].set(jnp.arange(N, dtype=_i32))
    dsr = rank[dst]
    order = jnp.argsort(dsr)
    dsr_s = dsr[order]
    eoff = jnp.searchsorted(dsr_s, noff).astype(_i32)
    esplit = jnp.searchsorted(dsr_s, NMID).astype(_i32)
    src_p = jnp.concatenate([rank[src][order],
                             jnp.zeros((E_PAD - E,), _i32)])
    dl0 = jnp.where(dsr_s < NMID, dsr_s, TRASH)
    dl1 = jnp.where(dsr_s >= NMID, dsr_s - NMID, TRASH)
    pad_tr = jnp.full((E_PAD - E,), TRASH, _i32)
    dst_p0 = jnp.concatenate([dl0, pad_tr])
    dst_p1 = jnp.concatenate([dl1, pad_tr])

    h0 = (W_emd[0] + b_emd).astype(_f32)
    h0r = h0[None, :]
    wat = W_aggr.T
    wimt = W_ih[:, :H].T
    wxt = W_ih[:, H:].T
    wht = W_hh.T
    bih = b_ih[None, :]
    bhh = b_hh[None, :]
    bag = b_aggr[None, :]
    b1r = b1[None, :]
    w2p = jnp.pad(W2, ((0, 0), (0, 128 - W2.shape[1])))
    b2r = jnp.pad(b2[None, :], ((0, 0), (0, 128 - b2.shape[0])))

    x_c = jnp.concatenate([x[node_order], jnp.zeros((NP - N, F), _f32)])

    zrow = jnp.zeros((RC, H), _f32)
    zdeg = jnp.zeros((RC, LANES), _f32)
    ones_h = jnp.ones((C, LANES), _f32)

    sc_params = []
    tc_params = []
    for l in range(1, L):
        n_lo_raw, n_hi = noff[l], noff[l + 1]
        e_a, e_c = eoff[l], eoff[l + 1]
        e_b = jnp.clip(esplit, e_a, e_c)
        lanes = []
        for (elo_raw, ehi, glo, ghi) in (
                (e_a, e_b, jnp.minimum(n_lo_raw, NMID), jnp.minimum(n_hi, NMID)),
                (e_b, e_c, jnp.maximum(n_lo_raw, NMID) - NMID,
                 jnp.maximum(n_hi, NMID) - NMID)):
            elo = _align8(elo_raw)
            ek = jnp.maximum(_cdiv(ehi - elo, C), 0)
            zlo = _align8(glo)
            nrch = jnp.maximum(_cdiv(ghi - zlo, RC), 0)
            lanes += [elo, _cdiv(ek, NS), ek, zlo, nrch, _cdiv(nrch, NS)]
        p = jnp.stack(lanes + [jnp.int32(0)] * (LANES - len(lanes))).astype(_i32)
        sc_params.append(p)
        first_blk = n_lo_raw // NBLK
        nblk = jnp.where(n_hi > n_lo_raw, (n_hi - 1) // NBLK - first_blk + 1, 0)
        tc_params.append(jnp.stack([first_blk, nblk, n_lo_raw, n_hi]).astype(_i32))

    xp, gh0 = _tc_pre(x_c, wxt, bih, h0r, wht, bhh)
    h_c = jnp.broadcast_to(h0r, (NP, H)) * jnp.ones((NP, 1), _f32)
    for i in range(L - 1):
        s_out, deg_out = _sc_level(sc_params[i], src_p, dst_p0, dst_p1, h_c,
                                   zrow, zdeg, ones_h)
        h_c = _tc_update(tc_params[i], s_out[:NP2], s_out[NP2:],
                         deg_out[:NP2], deg_out[NP2:], xp, h_c,
                         gh0, h0r, wat, wimt, bag)
    preds_full = _tc_mlp(h_c, W1, b1r, w2p, b2r)
    return preds_full[rank, :1]
